# Initial kernel scaffold; baseline (speedup 1.0000x reference)
import jax
import jax.numpy as jnp
from jax.experimental import pallas as pl


def kernel(x, edge_index, W1, b1, W2, b2, Wl, bl):
    raise NotImplementedError("write your pallas kernel here")


# speedup vs baseline: 2.3049x; 2.3049x over previous
import functools

import jax
import jax.numpy as jnp
from jax import lax
from jax.experimental import pallas as pl
from jax.experimental.pallas import tpu as pltpu
from jax.experimental.pallas import tpu_sc as plsc

NC = 2
NS = 16
LANES = 16
CHUNK = 128
ROWS_BLK = 2000
NH = 2
NR = 2


def _npad(n):
    return -(-(n + 1) // CHUNK) * CHUNK


def _npadx(n):
    return -(-(n + 1) // (NR * CHUNK)) * (NR * CHUNK)


def _deg_call(dst_chunks, n):
    n_chunks = dst_chunks.shape[0]
    cpt = n_chunks // (NC * NS)
    deg_rows = _npad(n)
    zslices = deg_rows // CHUNK
    zpt = -(-zslices // NS)
    wpt = deg_rows // NS

    @functools.partial(
        pl.kernel,
        out_type=jax.ShapeDtypeStruct((NC * deg_rows, LANES), jnp.float32),
        mesh=plsc.VectorSubcoreMesh(core_axis_name="c", subcore_axis_name="s"),
        scratch_types=[
            pltpu.VMEM((cpt, CHUNK), jnp.int32),
            pltpu.VMEM((CHUNK, LANES), jnp.float32),
            pltpu.VMEM((CHUNK, LANES), jnp.float32),
            pltpu.VMEM_SHARED((deg_rows, LANES), jnp.float32),
        ],
    )
    def deg_kernel(dst_hbm, out_hbm, idx_v, zeros_v, ones_v, deg_sh):
        c = lax.axis_index("c")
        s = lax.axis_index("s")
        tile = c * NS + s

        def fill(i, carry):
            zeros_v[i, :] = jnp.zeros((LANES,), jnp.float32)
            ones_v[i, :] = jnp.full((LANES,), 1.0, jnp.float32)
            return carry
        lax.fori_loop(0, CHUNK, fill, 0)

        pltpu.sync_copy(dst_hbm.at[pl.ds(tile * cpt, cpt)], idx_v)

        def zero(i, carry):
            j = s * zpt + i

            @pl.when(j < zslices)
            def _():
                pltpu.sync_copy(zeros_v, deg_sh.at[pl.ds(j * CHUNK, CHUNK)])
            return carry
        lax.fori_loop(0, zpt, zero, 0)
        plsc.subcore_barrier()

        def body(k, carry):
            pltpu.sync_copy(ones_v, deg_sh.at[idx_v.at[k]], add=True)
            return carry
        lax.fori_loop(0, cpt, body, 0)
        plsc.subcore_barrier()

        pltpu.sync_copy(deg_sh.at[pl.ds(s * wpt, wpt)],
                        out_hbm.at[pl.ds(c * deg_rows + s * wpt, wpt)])

    return deg_kernel(dst_chunks)


def _agg_call(gflat, src2_chunks, dstr_chunks, h, r):
    n_chunks = dstr_chunks.shape[0] // NR
    cpt = n_chunks // NS
    d = gflat.shape[1]
    npx = gflat.shape[0] // NH
    nph = npx // NR
    acc_rows = nph + 64
    wpt = nph // NS
    hoff = h * n_chunks
    roff = r * n_chunks

    @functools.partial(
        pl.kernel,
        out_type=jax.ShapeDtypeStruct((nph, d), jnp.float32),
        mesh=plsc.VectorSubcoreMesh(core_axis_name="c", subcore_axis_name="s",
                                    num_cores=1),
        scratch_types=[
            pltpu.VMEM((cpt, CHUNK), jnp.int32),
            pltpu.VMEM((cpt, CHUNK), jnp.int32),
            pltpu.VMEM((CHUNK, d), jnp.float32),
            pltpu.SemaphoreType.DMA,
            pltpu.VMEM_SHARED((acc_rows, d), jnp.float32),
        ],
    )
    def agg_kernel(g_hbm, src_hbm, dst_hbm, out_hbm,
                   src_v, dst_v, rows_v, sem, acc_sh):
        s = lax.axis_index("s")

        pltpu.sync_copy(src_hbm.at[pl.ds(hoff + s * cpt, cpt)], src_v)
        pltpu.sync_copy(dst_hbm.at[pl.ds(roff + s * cpt, cpt)], dst_v)
        pltpu.sync_copy(g_hbm.at[pl.ds(h * npx + r * nph + s * wpt, wpt)],
                        acc_sh.at[pl.ds(s * wpt, wpt)])
        plsc.subcore_barrier()

        def body(k, carry):
            pltpu.async_copy(g_hbm.at[src_v.at[k]], rows_v, sem).wait()
            pltpu.sync_copy(rows_v, acc_sh.at[dst_v.at[k]], add=True)
            return carry
        lax.fori_loop(0, cpt, body, 0)
        plsc.subcore_barrier()

        pltpu.sync_copy(acc_sh.at[pl.ds(s * wpt, wpt)],
                        out_hbm.at[pl.ds(s * wpt, wpt)])

    return agg_kernel(gflat, src2_chunks, dstr_chunks)


def _dinv(deg_ref):
    return lax.rsqrt(deg_ref[0][:, :1] + deg_ref[1][:, :1] + 1.0)


def _write_halves(g_ref, g):
    d = g.shape[1] // NH
    for q in range(NH):
        g_ref[q] = g[:, q * d:(q + 1) * d]


def _t1_body(deg_ref, x_ref, w_ref, g_ref):
    dinv = _dinv(deg_ref)
    h = jnp.dot(x_ref[...], w_ref[...], preferred_element_type=jnp.float32)
    _write_halves(g_ref, h * dinv)


def _t1_call(deg2, x, w1):
    n, f = x.shape
    h = w1.shape[1]
    return pl.pallas_call(
        _t1_body,
        grid=(n // ROWS_BLK,),
        in_specs=[
            pl.BlockSpec((2, ROWS_BLK, LANES), lambda i: (0, i, 0)),
            pl.BlockSpec((ROWS_BLK, f), lambda i: (i, 0)),
            pl.BlockSpec((f, h), lambda i: (0, 0)),
        ],
        out_specs=pl.BlockSpec((NH, ROWS_BLK, h // NH), lambda i: (0, i, 0)),
        out_shape=jax.ShapeDtypeStruct((NH, _npadx(n), h // NH), jnp.float32),
    )(deg2, x, w1)


def _relu_in(deg_ref, sa_ref, sb_ref, b_ref):
    dinv = _dinv(deg_ref)
    sfull = jnp.concatenate([sa_ref[...], sb_ref[...]], axis=1)
    return dinv, jnp.maximum(sfull * dinv + b_ref[...], 0.0)


def _t2_body(deg_ref, sa_ref, sb_ref, b_ref, w_ref, g_ref):
    dinv, o = _relu_in(deg_ref, sa_ref, sb_ref, b_ref)
    hh = jnp.dot(o, w_ref[...], preferred_element_type=jnp.float32)
    _write_halves(g_ref, hh * dinv)


def _t2_call(deg2, s1a, s1b, b1, w2, n):
    d = s1a.shape[1]
    h = w2.shape[1]
    return pl.pallas_call(
        _t2_body,
        grid=(n // ROWS_BLK,),
        in_specs=[
            pl.BlockSpec((2, ROWS_BLK, LANES), lambda i: (0, i, 0)),
            pl.BlockSpec((ROWS_BLK, d), lambda i: (i, 0)),
            pl.BlockSpec((ROWS_BLK, d), lambda i: (i, 0)),
            pl.BlockSpec((1, NH * d), lambda i: (0, 0)),
            pl.BlockSpec((NH * d, h), lambda i: (0, 0)),
        ],
        out_specs=pl.BlockSpec((NH, ROWS_BLK, h // NH), lambda i: (0, i, 0)),
        out_shape=jax.ShapeDtypeStruct((NH, _npadx(n), h // NH), jnp.float32),
    )(deg2, s1a, s1b, b1.reshape(1, -1), w2)


def _t3_body(deg_ref, sa_ref, sb_ref, b_ref, wl_ref, bl_ref, o_ref):
    _, o = _relu_in(deg_ref, sa_ref, sb_ref, b_ref)
    logits = jnp.dot(o, wl_ref[...], preferred_element_type=jnp.float32)
    logits = logits + bl_ref[...]
    m = jnp.max(logits, axis=1, keepdims=True)
    ex = jnp.exp(logits - m)
    lse = jnp.log(jnp.sum(ex, axis=1, keepdims=True))
    o_ref[...] = logits - m - lse


def _t3_call(deg2, s2a, s2b, b2, wl, bl, n):
    d = s2a.shape[1]
    k = wl.shape[1]
    return pl.pallas_call(
        _t3_body,
        grid=(n // ROWS_BLK,),
        in_specs=[
            pl.BlockSpec((2, ROWS_BLK, LANES), lambda i: (0, i, 0)),
            pl.BlockSpec((ROWS_BLK, d), lambda i: (i, 0)),
            pl.BlockSpec((ROWS_BLK, d), lambda i: (i, 0)),
            pl.BlockSpec((1, NH * d), lambda i: (0, 0)),
            pl.BlockSpec((NH * d, k), lambda i: (0, 0)),
            pl.BlockSpec((1, k), lambda i: (0, 0)),
        ],
        out_specs=pl.BlockSpec((ROWS_BLK, k), lambda i: (i, 0)),
        out_shape=jax.ShapeDtypeStruct((n, k), jnp.float32),
    )(deg2, s2a, s2b, b2.reshape(1, -1), wl, bl.reshape(1, -1))


def kernel(x, edge_index, W1, b1, W2, b2, Wl, bl):
    n = x.shape[0]
    e = edge_index.shape[1]
    ei = edge_index.astype(jnp.int32)
    npx = _npadx(n)
    nph = npx // NR
    egrain = CHUNK * NC * NS * 8
    epad = -(-e // egrain) * egrain
    pad = epad - e
    src = jnp.concatenate([ei[0], jnp.zeros((pad,), jnp.int32)])
    dst = jnp.concatenate([ei[1], jnp.full((pad,), n, jnp.int32)])
    dst_chunks = dst.reshape(-1, CHUNK)
    src2 = jnp.concatenate([src + q * npx for q in range(NH)]).reshape(-1, CHUNK)
    dstr = jnp.concatenate(
        [jnp.where((dst >= r * nph) & (dst < (r + 1) * nph), dst - r * nph, nph)
         for r in range(NR)]).reshape(-1, CHUNK)

    deg2 = _deg_call(dst_chunks, n).reshape(NC, -1, LANES)

    def agg_layer(gh):
        gflat = gh.reshape(NH * npx, -1)
        return [jnp.concatenate(
                    [_agg_call(gflat, src2, dstr, h, r) for r in range(NR)])
                for h in range(NH)]

    g1 = _t1_call(deg2, x, W1)
    s1a, s1b = agg_layer(g1)
    g2 = _t2_call(deg2, s1a, s1b, b1, W2, n)
    s2a, s2b = agg_layer(g2)
    out = _t3_call(deg2, s2a, s2b, b2, Wl, bl, n)
    return (out, edge_index)
